# baseline (speedup 1.0000x reference)
import functools

import jax
import jax.numpy as jnp
from jax.experimental import pallas as pl
from jax.experimental.pallas import tpu as pltpu

B = 256
IN_DIM = 512
OUT_DIM = 10000
BN = 256
BS = 2048


def _support_body(x_ref, w_ref, s_ref):
    s_ref[...] = jax.lax.dot_general(
        x_ref[...],
        w_ref[...],
        dimension_numbers=(((1,), (0,)), ((), ())),
        preferred_element_type=jnp.float32,
    )


def _agg_body(adj_ref, s_ref, bias_ref, out_ref):
    acc = jax.lax.dot_general(
        s_ref[...],
        adj_ref[...],
        dimension_numbers=(((1,), (1,)), ((), ())),
        preferred_element_type=jnp.float32,
        precision=jax.lax.Precision.DEFAULT,
    )
    out_ref[...] = acc + bias_ref[...]


@functools.partial(jax.jit, static_argnames=())
def kernel(input, adj, weight, bias):
    bias2d = bias.reshape(1, OUT_DIM)

    support = pl.pallas_call(
        _support_body,
        grid=(pl.cdiv(OUT_DIM, BS),),
        in_specs=[
            pl.BlockSpec((B, IN_DIM), lambda n: (0, 0)),
            pl.BlockSpec((IN_DIM, BS), lambda n: (0, n)),
        ],
        out_specs=pl.BlockSpec((B, BS), lambda n: (0, n)),
        out_shape=jax.ShapeDtypeStruct((B, OUT_DIM), jnp.float32),
        compiler_params=pltpu.CompilerParams(
            dimension_semantics=("parallel",),
        ),
    )(input, weight)

    out = pl.pallas_call(
        _agg_body,
        grid=(pl.cdiv(OUT_DIM, BN),),
        in_specs=[
            pl.BlockSpec((BN, OUT_DIM), lambda n: (n, 0)),
            pl.BlockSpec((B, OUT_DIM), lambda n: (0, 0)),
            pl.BlockSpec((1, BN), lambda n: (0, n)),
        ],
        out_specs=pl.BlockSpec((B, BN), lambda n: (0, n)),
        out_shape=jax.ShapeDtypeStruct((B, OUT_DIM), jnp.float32),
        compiler_params=pltpu.CompilerParams(
            dimension_semantics=("parallel",),
        ),
    )(adj, support, bias2d)
    return out


# speedup vs baseline: 1.0621x; 1.0621x over previous
import functools

import jax
import jax.numpy as jnp
from jax.experimental import pallas as pl
from jax.experimental.pallas import tpu as pltpu

B = 256
IN_DIM = 512
OUT_DIM = 10000
BN = 256


def _agg_body(adj_ref, xt_ref, w_ref, bias_ref, out_ref, st_ref):
    @pl.when(pl.program_id(0) == 0)
    def _():
        st_ref[...] = jax.lax.dot_general(
            w_ref[...],
            xt_ref[...],
            dimension_numbers=(((0,), (0,)), ((), ())),
            preferred_element_type=jnp.float32,
        )

    acc = jax.lax.dot_general(
        adj_ref[...],
        st_ref[...],
        dimension_numbers=(((1,), (0,)), ((), ())),
        preferred_element_type=jnp.float32,
        precision=jax.lax.Precision.DEFAULT,
    )
    out_ref[...] = acc + bias_ref[...]


@functools.partial(jax.jit, static_argnames=())
def kernel(input, adj, weight, bias):
    xt = input.T
    bias_col = bias.reshape(OUT_DIM, 1)

    out_t = pl.pallas_call(
        _agg_body,
        grid=(pl.cdiv(OUT_DIM, BN),),
        in_specs=[
            pl.BlockSpec((BN, OUT_DIM), lambda n: (n, 0)),
            pl.BlockSpec((IN_DIM, B), lambda n: (0, 0)),
            pl.BlockSpec((IN_DIM, OUT_DIM), lambda n: (0, 0)),
            pl.BlockSpec((BN, 1), lambda n: (n, 0)),
        ],
        out_specs=pl.BlockSpec((BN, B), lambda n: (n, 0)),
        out_shape=jax.ShapeDtypeStruct((OUT_DIM, B), jnp.float32),
        scratch_shapes=[
            pltpu.VMEM((OUT_DIM, B), jnp.float32),
        ],
        compiler_params=pltpu.CompilerParams(
            dimension_semantics=("arbitrary",),
            vmem_limit_bytes=110 * 1024 * 1024,
        ),
    )(adj, xt, weight, bias_col)
    return out_t.T
